# baseline (speedup 1.0000x reference)
import functools

import jax
import jax.numpy as jnp
from jax import lax
from jax.experimental import pallas as pl
from jax.experimental.pallas import tpu as pltpu

E = 8
K = 2
T = 2048
D = 2048
H = 2048

BT = 128
NB = (T * K) // BT + E
P = NB * BT


def _ffn_block(gid_ref, x_ref, up_ref, down_ref, out_ref):
    x = x_ref[...]
    up = lax.dot_general(x, up_ref[0], (((1,), (1,)), ((), ())),
                         preferred_element_type=jnp.float32)
    g = up[:, :H]
    l = up[:, H:]
    act = ((g * jax.nn.sigmoid(g)) * l).astype(jnp.bfloat16)
    out_ref[...] = lax.dot_general(act, down_ref[0], (((1,), (1,)), ((), ())),
                                   preferred_element_type=jnp.float32
                                   ).astype(jnp.bfloat16)


def _grouped_ffn(x_sorted, gid, up_bf, down_bf):
    spec = pltpu.PrefetchScalarGridSpec(
        num_scalar_prefetch=1,
        grid=(NB,),
        in_specs=[
            pl.BlockSpec((BT, D), lambda b, gid: (b, 0)),
            pl.BlockSpec((1, 2 * H, D), lambda b, gid: (gid[b], 0, 0)),
            pl.BlockSpec((1, D, H), lambda b, gid: (gid[b], 0, 0)),
        ],
        out_specs=pl.BlockSpec((BT, D), lambda b, gid: (b, 0)),
    )
    return pl.pallas_call(
        _ffn_block,
        grid_spec=spec,
        out_shape=jax.ShapeDtypeStruct((P, D), jnp.bfloat16),
    )(gid, x_sorted, up_bf, down_bf)


def kernel(hidden_states, topk_weights, topk_ids, up_weight, down_weight):
    ids = topk_ids.T.reshape(-1)
    counts = jnp.bincount(ids, length=E)
    starts = jnp.concatenate([jnp.zeros((1,), jnp.int32),
                              jnp.cumsum(counts)[:-1].astype(jnp.int32)])
    nb = (counts + BT - 1) // BT
    cum_nb = jnp.cumsum(nb)
    aligned_off = BT * jnp.concatenate(
        [jnp.zeros((1,), jnp.int32), cum_nb[:-1].astype(jnp.int32)])
    order = jnp.argsort(ids, stable=True)
    inv = jnp.argsort(order)
    rank_in_e = inv - starts[ids]
    pos = aligned_off[ids] + rank_in_e

    b_iota = jnp.arange(NB)
    gid = jnp.minimum(
        jnp.sum(b_iota[:, None] >= cum_nb[None, :], axis=1), E - 1
    ).astype(jnp.int32)

    tok = jnp.arange(T * K) % T
    x_sorted = (jnp.zeros((P, D), jnp.bfloat16)
                .at[pos].set(hidden_states[tok].astype(jnp.bfloat16)))

    up_bf = up_weight.astype(jnp.bfloat16)
    down_bf = down_weight.astype(jnp.bfloat16)

    d = _grouped_ffn(x_sorted, gid, up_bf, down_bf)

    pos2 = pos.reshape(K, T)
    out = jnp.zeros((T, D), jnp.float32)
    for k in range(K):
        out = out + topk_weights[:, k][:, None] * d[pos2[k]].astype(jnp.float32)
    return out
